# baseline (speedup 1.0000x reference)
import functools

import jax
import jax.numpy as jnp
from jax import lax
from jax.experimental import pallas as pl
from jax.experimental.pallas import tpu as pltpu
from jax.experimental.pallas import tpu_sc as plsc

NUM_ITEM = 1000000
NUM_YEAR = 100
NUM_GENRE = 25
EMBED_DIM = 32
BATCH = 16384
OUT_DIM = 3 * EMBED_DIM
LANES = 128

NC = 2
NS = 16
L = 16
NW = NC * NS
BPW = BATCH // NW
CHUNKS = BPW // L

NIB = 5
NNIB = NUM_GENRE // NIB
NCOMBO = 1 << NIB


def _sc_body(item_idx, year_idx, gflat, item_tab, ypad, w_flat, out,
             iv, yv, gb_v, w_v, yt_v, t_v, item_rows, out_buf,
             sem_aux, sem_main):
    wid = lax.axis_index("s") * NC + lax.axis_index("c")
    base = wid * BPW
    iota = lax.iota(jnp.int32, L)

    aux = [
        pltpu.async_copy(year_idx.at[pl.ds(base, BPW)], yv, sem_aux),
        pltpu.async_copy(w_flat, w_v, sem_aux),
        pltpu.async_copy(ypad, yt_v, sem_aux),
    ]
    for g in range(NUM_GENRE):
        aux.append(pltpu.async_copy(
            gflat.at[pl.ds(g * BATCH + base, BPW)],
            gb_v.at[pl.ds(g * BPW, BPW)], sem_aux))

    pltpu.sync_copy(item_idx.at[pl.ds(base, BPW)], iv)
    cp_main = pltpu.async_copy(item_tab.at[iv], item_rows, sem_main)

    for a in aux:
        a.wait()

    zeros = jnp.zeros((L,), jnp.float32)
    for n in range(NNIB):
        tb = (n * NCOMBO) * EMBED_DIM
        t_v[pl.ds(tb, L)] = zeros
        t_v[pl.ds(tb + L, L)] = zeros
        acc_lo, acc_hi = zeros, zeros
        for k in range(1, NCOMBO):
            g = k ^ (k >> 1)
            prev = (k - 1) ^ ((k - 1) >> 1)
            b = (g ^ prev).bit_length() - 1
            gg = NIB * n + b
            w_lo = w_v[pl.ds(gg * EMBED_DIM, L)]
            w_hi = w_v[pl.ds(gg * EMBED_DIM + L, L)]
            if g & (1 << b):
                acc_lo, acc_hi = acc_lo + w_lo, acc_hi + w_hi
            else:
                acc_lo, acc_hi = acc_lo - w_lo, acc_hi - w_hi
            t_v[pl.ds(tb + g * EMBED_DIM, L)] = acc_lo
            t_v[pl.ds(tb + g * EMBED_DIM + L, L)] = acc_hi

    def genre_chunk(c, carry):
        cnt = jnp.zeros((L,), jnp.int32)
        tidx = []
        for n in range(NNIB):
            cb = jnp.zeros((L,), jnp.int32)
            for b in range(NIB):
                bits = gb_v[pl.ds((NIB * n + b) * BPW + c * L, L)]
                cnt = cnt + bits
                cb = cb + (bits << b) if b else bits
            tidx.append((n * NCOMBO) * EMBED_DIM + cb * EMBED_DIM)
        inv = 1.0 / jnp.maximum(cnt.astype(jnp.float32), 1.0)
        pos = (c * L + iota) * OUT_DIM + 2 * EMBED_DIM
        for j in range(EMBED_DIM):
            v = plsc.load_gather(t_v, [tidx[0] + j])
            for n in range(1, NNIB):
                v = v + plsc.load_gather(t_v, [tidx[n] + j])
            plsc.store_scatter(out_buf, [pos + j], v * inv)
        return carry
    lax.fori_loop(0, CHUNKS, genre_chunk, 0)

    def year_chunk(c, carry):
        ycvec = yv[pl.ds(c * L, L)]
        pos = (c * L + iota) * OUT_DIM + EMBED_DIM
        for j in range(EMBED_DIM):
            vals = plsc.load_gather(
                yt_v, [jnp.full((L,), j, jnp.int32), ycvec])
            plsc.store_scatter(out_buf, [pos + j], vals)
        return carry
    lax.fori_loop(0, CHUNKS, year_chunk, 0)

    cp_main.wait()

    def weave_chunk(c, carry):
        rid = c * L + iota
        pos = rid * OUT_DIM
        for j in range(EMBED_DIM):
            vals = plsc.load_gather(
                item_rows, [rid, jnp.full((L,), j, jnp.int32)])
            plsc.store_scatter(out_buf, [pos + j], vals)
        return carry
    lax.fori_loop(0, CHUNKS, weave_chunk, 0)

    pltpu.sync_copy(out_buf, out.at[pl.ds(base * OUT_DIM, BPW * OUT_DIM)])


@jax.jit
def _run(item_idx, year_idx, gflat, item_tab, ypad, w_flat):
    mesh = plsc.VectorSubcoreMesh(core_axis_name="c", subcore_axis_name="s")
    f = pl.kernel(
        _sc_body,
        out_type=jax.ShapeDtypeStruct((BATCH * OUT_DIM,), jnp.float32),
        mesh=mesh,
        scratch_types=[
            pltpu.VMEM((BPW,), jnp.int32),
            pltpu.VMEM((BPW,), jnp.int32),
            pltpu.VMEM((NUM_GENRE * BPW,), jnp.int32),
            pltpu.VMEM((NUM_GENRE * EMBED_DIM,), jnp.float32),
            pltpu.VMEM((EMBED_DIM, LANES), jnp.float32),
            pltpu.VMEM((NNIB * NCOMBO * EMBED_DIM,), jnp.float32),
            pltpu.VMEM((BPW, EMBED_DIM), jnp.float32),
            pltpu.VMEM((BPW * OUT_DIM,), jnp.float32),
            pltpu.SemaphoreType.DMA,
            pltpu.SemaphoreType.DMA,
        ],
        compiler_params=pltpu.CompilerParams(
            use_tc_tiling_on_sc=False, needs_layout_passes=False),
    )
    return f(item_idx, year_idx, gflat, item_tab, ypad, w_flat)


def kernel(item_fea, item_embedding, year_embedding, genre_embedding):
    fea = item_fea.astype(jnp.int32)
    item_idx = fea[:, 0]
    year_idx = fea[:, 1]
    gflat = fea[:, 2:2 + NUM_GENRE].T.reshape(-1)
    ypad = jnp.pad(year_embedding.T, ((0, 0), (0, LANES - NUM_YEAR)))
    w_flat = genre_embedding.T.reshape(-1)
    out = _run(item_idx, year_idx, gflat, item_embedding, ypad, w_flat)
    return out.reshape(BATCH, OUT_DIM)
